# Initial kernel scaffold; baseline (speedup 1.0000x reference)
import jax
import jax.numpy as jnp
from jax.experimental import pallas as pl


def kernel(x_user, x_item, edge_label_index):
    raise NotImplementedError("write your pallas kernel here")


# speedup vs baseline: 7.4702x; 7.4702x over previous
import jax
import jax.numpy as jnp
from jax import lax
from jax.experimental import pallas as pl
from jax.experimental.pallas import tpu as pltpu, tpu_sc as plsc

N_ROWS = 100000
D = 64
E = 1048576
L = 16
NC, NS = 2, 16
NW = NC * NS
PER_W = E // NW
C = 128
NCHUNKS = PER_W // C


def _body(xu_hbm, xi_hbm, src_hbm, dst_hbm, out_hbm,
          idx_u, idx_i, urows, irows, outv, tsc, sem_u, sem_i):
    wid = lax.axis_index("s") * NC + lax.axis_index("c")
    wbase = wid * PER_W

    def chunk_body(ci, carry):
        base = wbase + ci * C
        pltpu.sync_copy(src_hbm.at[pl.ds(base, C)], idx_u)
        pltpu.sync_copy(dst_hbm.at[pl.ds(base, C)], idx_i)
        cu = pltpu.async_copy(xu_hbm.at[idx_u], urows, sem_u)
        cd = pltpu.async_copy(xi_hbm.at[idx_i], irows, sem_i)
        cu.wait()
        cd.wait()

        lane = lax.broadcasted_iota(jnp.int32, (L,), 0)

        def group_body(g, c2):
            ebase = g * L
            for j in range(L):
                e = ebase + j
                acc = urows[e, pl.ds(0, L)] * irows[e, pl.ds(0, L)]
                for dd in range(1, D // L):
                    acc = acc + urows[e, pl.ds(dd * L, L)] * irows[e, pl.ds(dd * L, L)]
                plsc.store_scatter(tsc, [lane * L + j], acc)
            res = tsc[pl.ds(0, L)]
            for l in range(1, L):
                res = res + tsc[pl.ds(l * L, L)]
            outv[pl.ds(ebase, L)] = res
            return c2

        lax.fori_loop(0, C // L, group_body, 0)
        pltpu.sync_copy(outv, out_hbm.at[pl.ds(base, C)])
        return carry

    lax.fori_loop(0, NCHUNKS, chunk_body, 0)


def kernel(x_user, x_item, edge_label_index):
    src = edge_label_index[0]
    dst = edge_label_index[1]
    mesh = plsc.VectorSubcoreMesh(core_axis_name="c", subcore_axis_name="s",
                                  num_cores=NC, num_subcores=NS)
    f = pl.kernel(
        _body,
        out_type=jax.ShapeDtypeStruct((E,), jnp.float32),
        mesh=mesh,
        scratch_types=[
            pltpu.VMEM((C,), jnp.int32),
            pltpu.VMEM((C,), jnp.int32),
            pltpu.VMEM((C, D), jnp.float32),
            pltpu.VMEM((C, D), jnp.float32),
            pltpu.VMEM((C,), jnp.float32),
            pltpu.VMEM((L * L,), jnp.float32),
            pltpu.SemaphoreType.DMA,
            pltpu.SemaphoreType.DMA,
        ],
        compiler_params=pltpu.CompilerParams(needs_layout_passes=False,
                                             use_tc_tiling_on_sc=False),
    )
    return f(x_user, x_item, src, dst)


# speedup vs baseline: 12.2659x; 1.6420x over previous
import jax
import jax.numpy as jnp
from jax import lax
from jax.experimental import pallas as pl
from jax.experimental.pallas import tpu as pltpu, tpu_sc as plsc

D = 64
E = 1048576
L = 16
NC, NS = 2, 16
NW = NC * NS
PER_W = E // NW
C = 128
NBUF = 4
P = NBUF - 1
IDXB = 2048
CPB = IDXB // C
NBLK = PER_W // IDXB


def _body(xu_hbm, xi_hbm, src_hbm, dst_hbm, out_hbm,
          idx_u, idx_i, urows, irows, outv, tsc,
          su0, su1, su2, su3, si0, si1, si2, si3):
    sem_u = [su0, su1, su2, su3]
    sem_i = [si0, si1, si2, si3]
    wid = lax.axis_index("s") * NC + lax.axis_index("c")
    wbase = wid * PER_W
    wrow = wid * (PER_W // C)
    lane = lax.broadcasted_iota(jnp.int32, (L,), 0)

    def block_body(k, carry):
        brow = wrow + k * CPB
        pltpu.sync_copy(src_hbm.at[pl.ds(brow, CPB), :], idx_u)
        pltpu.sync_copy(dst_hbm.at[pl.ds(brow, CPB), :], idx_i)

        def start(j):
            s = j % NBUF
            du = pltpu.async_copy(xu_hbm.at[idx_u.at[j]], urows.at[s], sem_u[s])
            di = pltpu.async_copy(xi_hbm.at[idx_i.at[j]], irows.at[s], sem_i[s])
            return (du, di)

        desc = {}
        for p in range(P):
            desc[p % NBUF] = start(p)

        for j in range(CPB):
            s = j % NBUF
            if j + P < CPB:
                desc[(j + P) % NBUF] = start(j + P)
            du, di = desc[s]
            du.wait()
            di.wait()
            obase = (k * CPB + j) * C

            def group_body(g, c2):
                ebase = g * L
                for jj in range(L):
                    e = ebase + jj
                    acc = urows[s, e, pl.ds(0, L)] * irows[s, e, pl.ds(0, L)]
                    for dd in range(1, D // L):
                        acc = acc + (urows[s, e, pl.ds(dd * L, L)]
                                     * irows[s, e, pl.ds(dd * L, L)])
                    plsc.store_scatter(tsc, [lane * L + jj], acc)
                res = tsc[pl.ds(0, L)]
                for l in range(1, L):
                    res = res + tsc[pl.ds(l * L, L)]
                outv[pl.ds(obase + ebase, L)] = res
                return c2

            lax.fori_loop(0, C // L, group_body, 0)
        return carry

    lax.fori_loop(0, NBLK, block_body, 0)
    pltpu.sync_copy(outv, out_hbm.at[pl.ds(wbase, PER_W)])


def kernel(x_user, x_item, edge_label_index):
    src = edge_label_index[0].reshape(E // C, C)
    dst = edge_label_index[1].reshape(E // C, C)
    mesh = plsc.VectorSubcoreMesh(core_axis_name="c", subcore_axis_name="s",
                                  num_cores=NC, num_subcores=NS)
    f = pl.kernel(
        _body,
        out_type=jax.ShapeDtypeStruct((E,), jnp.float32),
        mesh=mesh,
        scratch_types=[
            pltpu.VMEM((CPB, C), jnp.int32),
            pltpu.VMEM((CPB, C), jnp.int32),
            pltpu.VMEM((NBUF, C, D), jnp.float32),
            pltpu.VMEM((NBUF, C, D), jnp.float32),
            pltpu.VMEM((PER_W,), jnp.float32),
            pltpu.VMEM((L * L,), jnp.float32),
        ] + [pltpu.SemaphoreType.DMA] * (2 * NBUF),
        compiler_params=pltpu.CompilerParams(needs_layout_passes=False,
                                             use_tc_tiling_on_sc=False),
    )
    return f(x_user, x_item, src, dst)


# speedup vs baseline: 16.6946x; 1.3611x over previous
import jax
import jax.numpy as jnp
from jax import lax
from jax.experimental import pallas as pl
from jax.experimental.pallas import tpu as pltpu, tpu_sc as plsc

D = 64
E = 1048576
L = 16
NC, NS = 2, 16
NW = NC * NS
PER_W = E // NW
C = 128
G = C // L
NBUF = 4
P = NBUF - 1
IDXB = 2048
CPB = IDXB // C
NBLK = PER_W // IDXB


def _body(xu_hbm, xi_hbm, eli_hbm, out_hbm,
          idx_u, idx_i, urows, irows, outv, tsc,
          su0, su1, su2, su3, si0, si1, si2, si3):
    sem_u = [su0, su1, su2, su3]
    sem_i = [si0, si1, si2, si3]
    wid = lax.axis_index("s") * NC + lax.axis_index("c")
    wbase = wid * PER_W
    wrow = wid * (PER_W // C)
    lane = lax.broadcasted_iota(jnp.int32, (L,), 0)

    def block_body(k, carry):
        brow = wrow + k * CPB
        pltpu.sync_copy(eli_hbm.at[0, pl.ds(brow, CPB), :], idx_u)
        pltpu.sync_copy(eli_hbm.at[1, pl.ds(brow, CPB), :], idx_i)

        def start(j):
            s = j % NBUF
            du = pltpu.async_copy(xu_hbm.at[idx_u.at[j]], urows.at[s], sem_u[s])
            di = pltpu.async_copy(xi_hbm.at[idx_i.at[j]], irows.at[s], sem_i[s])
            return (du, di)

        desc = {}
        for p in range(P):
            desc[p % NBUF] = start(p)

        for j in range(CPB):
            s = j % NBUF
            if j + P < CPB:
                desc[(j + P) % NBUF] = start(j + P)
            du, di = desc[s]
            du.wait()
            di.wait()
            obase = (k * CPB + j) * C

            @plsc.parallel_loop(0, G, unroll=2)
            def group_body(g):
                ebase = g * L
                tbase = g * (L * L)
                for jj in range(L):
                    e = ebase + jj
                    m0 = urows[s, e, pl.ds(0, L)] * irows[s, e, pl.ds(0, L)]
                    m1 = urows[s, e, pl.ds(L, L)] * irows[s, e, pl.ds(L, L)]
                    m2 = urows[s, e, pl.ds(2 * L, L)] * irows[s, e, pl.ds(2 * L, L)]
                    m3 = urows[s, e, pl.ds(3 * L, L)] * irows[s, e, pl.ds(3 * L, L)]
                    acc = (m0 + m1) + (m2 + m3)
                    plsc.store_scatter(tsc, [tbase + lane * L + jj], acc)
                t = [tsc[pl.ds(tbase + l * L, L)] for l in range(L)]
                while len(t) > 1:
                    t = [t[2 * i] + t[2 * i + 1] for i in range(len(t) // 2)]
                outv[pl.ds(obase + ebase, L)] = t[0]

        return carry

    lax.fori_loop(0, NBLK, block_body, 0)
    pltpu.sync_copy(outv, out_hbm.at[pl.ds(wbase, PER_W)])


def kernel(x_user, x_item, edge_label_index):
    eli = edge_label_index.reshape(2, E // C, C)
    mesh = plsc.VectorSubcoreMesh(core_axis_name="c", subcore_axis_name="s",
                                  num_cores=NC, num_subcores=NS)
    f = pl.kernel(
        _body,
        out_type=jax.ShapeDtypeStruct((E,), jnp.float32),
        mesh=mesh,
        scratch_types=[
            pltpu.VMEM((CPB, C), jnp.int32),
            pltpu.VMEM((CPB, C), jnp.int32),
            pltpu.VMEM((NBUF, C, D), jnp.float32),
            pltpu.VMEM((NBUF, C, D), jnp.float32),
            pltpu.VMEM((PER_W,), jnp.float32),
            pltpu.VMEM((G * L * L,), jnp.float32),
        ] + [pltpu.SemaphoreType.DMA] * (2 * NBUF),
        compiler_params=pltpu.CompilerParams(needs_layout_passes=False,
                                             use_tc_tiling_on_sc=False),
    )
    return f(x_user, x_item, eli)
